# baseline (speedup 1.0000x reference)
import functools

import jax
import jax.numpy as jnp
from jax import lax
from jax.experimental import pallas as pl
from jax.experimental.pallas import tpu as pltpu
from jax.experimental.pallas import tpu_sc as plsc

EMBED_D = 128
NUM_WORKERS = 32
GATHER_ROWS = 128
TABLE_PAD = 1024
NBUF = 4


def _make_sc_gather(num_rows: int):
    rows_per_w = num_rows // NUM_WORKERS
    chunks = rows_per_w // GATHER_ROWS
    assert chunks % NBUF == 0
    slab = TABLE_PAD // 16

    mesh = plsc.VectorSubcoreMesh(core_axis_name="c", subcore_axis_name="s")

    @functools.partial(
        pl.kernel,
        mesh=mesh,
        out_type=jax.ShapeDtypeStruct((num_rows, EMBED_D), jnp.float32),
        scratch_types=[
            pltpu.VMEM_SHARED((TABLE_PAD, EMBED_D), jnp.float32),
            pltpu.VMEM((chunks, GATHER_ROWS), jnp.int32),
        ]
        + [pltpu.VMEM((GATHER_ROWS, EMBED_D), jnp.float32)] * NBUF
        + [pltpu.SemaphoreType.DMA] * (2 * NBUF),
    )
    def k(ids_hbm, w_hbm, out_hbm, table_sh, idx_v, *rest):
        bufs = rest[:NBUF]
        gsem = rest[NBUF:2 * NBUF]
        ssem = rest[2 * NBUF:]
        cid = lax.axis_index("c")
        sid = lax.axis_index("s")
        wid = sid * 2 + cid
        base = wid * rows_per_w

        pltpu.sync_copy(w_hbm.at[pl.ds(sid * slab, slab)],
                        table_sh.at[pl.ds(sid * slab, slab)])
        pltpu.sync_copy(ids_hbm.at[wid], idx_v)
        plsc.subcore_barrier()

        def gather(b, j):
            return pltpu.make_async_copy(table_sh.at[idx_v.at[j]],
                                         bufs[b], gsem[b])

        def store(b, j):
            return pltpu.make_async_copy(
                bufs[b], out_hbm.at[pl.ds(base + j * GATHER_ROWS,
                                          GATHER_ROWS)], ssem[b])

        gather(0, 0).start()
        gather(1, 1).start()

        def body(i, carry):
            for u in range(NBUF):
                j = i * NBUF + u
                b = u
                gather(b, j).wait()
                store(b, j).start()
                b2 = (u + 2) % NBUF

                @pl.when(j + 2 < chunks)
                def _():
                    @pl.when(j >= 2)
                    def _():
                        store(b2, j - 2).wait()

                    gather(b2, j + 2).start()

            return carry

        lax.fori_loop(0, chunks // NBUF, body, 0, unroll=False)

        for u in range(NBUF):
            j = chunks - NBUF + u
            store(u, j).wait()

    return k


def kernel(sensor_ids, weight):
    b, t = sensor_ids.shape
    num_rows = b * t
    ids = sensor_ids.astype(jnp.int32).reshape(
        NUM_WORKERS, num_rows // (NUM_WORKERS * GATHER_ROWS), GATHER_ROWS)
    w_pad = jnp.pad(weight, ((0, TABLE_PAD - weight.shape[0]), (0, 0)))
    out = _make_sc_gather(num_rows)(ids, w_pad)
    return out.reshape(b, t, EMBED_D)


# speedup vs baseline: 1.0081x; 1.0081x over previous
import functools

import jax
import jax.numpy as jnp
from jax import lax
from jax.experimental import pallas as pl
from jax.experimental.pallas import tpu as pltpu
from jax.experimental.pallas import tpu_sc as plsc

EMBED_D = 128
NUM_WORKERS = 32
GATHER_ROWS = 128
TABLE_PAD = 1024
NBUF = 5
DEPTH = 3


def _make_sc_gather(num_rows: int):
    rows_per_w = num_rows // NUM_WORKERS
    chunks = rows_per_w // GATHER_ROWS
    assert chunks % NBUF == 0
    slab = TABLE_PAD // 16

    mesh = plsc.VectorSubcoreMesh(core_axis_name="c", subcore_axis_name="s")

    @functools.partial(
        pl.kernel,
        mesh=mesh,
        out_type=jax.ShapeDtypeStruct((num_rows, EMBED_D), jnp.float32),
        scratch_types=[
            pltpu.VMEM_SHARED((TABLE_PAD, EMBED_D), jnp.float32),
            pltpu.VMEM((chunks, GATHER_ROWS), jnp.int32),
        ]
        + [pltpu.VMEM((GATHER_ROWS, EMBED_D), jnp.float32)] * NBUF
        + [pltpu.SemaphoreType.DMA] * (2 * NBUF),
    )
    def k(ids_hbm, w_hbm, out_hbm, table_sh, idx_v, *rest):
        bufs = rest[:NBUF]
        gsem = rest[NBUF:2 * NBUF]
        ssem = rest[2 * NBUF:]
        cid = lax.axis_index("c")
        sid = lax.axis_index("s")
        wid = sid * 2 + cid
        base = wid * rows_per_w

        pltpu.sync_copy(w_hbm.at[pl.ds(sid * slab, slab)],
                        table_sh.at[pl.ds(sid * slab, slab)])
        pltpu.sync_copy(ids_hbm.at[wid], idx_v)
        plsc.subcore_barrier()

        def gather(b, j):
            return pltpu.make_async_copy(table_sh.at[idx_v.at[j]],
                                         bufs[b], gsem[b])

        def store(b, j):
            return pltpu.make_async_copy(
                bufs[b], out_hbm.at[pl.ds(base + j * GATHER_ROWS,
                                          GATHER_ROWS)], ssem[b])

        for p in range(DEPTH):
            gather(p, p).start()

        def body(i, carry):
            for u in range(NBUF):
                j = i * NBUF + u
                b = u
                gather(b, j).wait()
                store(b, j).start()
                b2 = (u + DEPTH) % NBUF

                @pl.when(j + DEPTH < chunks)
                def _():
                    @pl.when(j >= NBUF - DEPTH)
                    def _():
                        store(b2, j + DEPTH - NBUF).wait()

                    gather(b2, j + DEPTH).start()

            return carry

        lax.fori_loop(0, chunks // NBUF, body, 0, unroll=False)

        for u in range(NBUF):
            j = chunks - NBUF + u
            store(u, j).wait()

    return k


def kernel(sensor_ids, weight):
    b, t = sensor_ids.shape
    num_rows = b * t
    ids = sensor_ids.astype(jnp.int32).reshape(
        NUM_WORKERS, num_rows // (NUM_WORKERS * GATHER_ROWS), GATHER_ROWS)
    w_pad = jnp.pad(weight, ((0, TABLE_PAD - weight.shape[0]), (0, 0)))
    out = _make_sc_gather(num_rows)(ids, w_pad)
    return out.reshape(b, t, EMBED_D)


# speedup vs baseline: 1.0083x; 1.0002x over previous
import functools

import jax
import jax.numpy as jnp
from jax import lax
from jax.experimental import pallas as pl
from jax.experimental.pallas import tpu as pltpu
from jax.experimental.pallas import tpu_sc as plsc

EMBED_D = 128
NUM_WORKERS = 32
GATHER_ROWS = 128
TABLE_PAD = 1024
NBUF = 5
DEPTH = 3


def _make_sc_gather(num_rows: int):
    rows_per_w = num_rows // NUM_WORKERS
    chunks = rows_per_w // GATHER_ROWS
    assert chunks % NBUF == 0
    slab = TABLE_PAD // 16

    mesh = plsc.VectorSubcoreMesh(core_axis_name="c", subcore_axis_name="s")

    @functools.partial(
        pl.kernel,
        mesh=mesh,
        out_type=jax.ShapeDtypeStruct((num_rows, EMBED_D), jnp.float32),
        scratch_types=[
            pltpu.VMEM_SHARED((TABLE_PAD, EMBED_D), jnp.float32),
            pltpu.VMEM((chunks, GATHER_ROWS), jnp.int32),
        ]
        + [pltpu.VMEM((4 * GATHER_ROWS, EMBED_D), jnp.float32)]
        + [pltpu.SemaphoreType.DMA] * 6,
    )
    def k(ids_hbm, w_hbm, out_hbm, table_sh, idx_v, bufs, *rest):
        gsem = rest[:4]
        ssem = rest[4:]
        cid = lax.axis_index("c")
        sid = lax.axis_index("s")
        wid = sid * 2 + cid
        base = wid * rows_per_w

        pltpu.sync_copy(w_hbm.at[pl.ds(sid * slab, slab)],
                        table_sh.at[pl.ds(sid * slab, slab)])
        pltpu.sync_copy(ids_hbm.at[wid], idx_v)
        plsc.subcore_barrier()

        pairs = chunks // 2

        def gather(slot, j):
            return pltpu.make_async_copy(
                table_sh.at[idx_v.at[j]],
                bufs.at[pl.ds(slot * GATHER_ROWS, GATHER_ROWS)],
                gsem[slot])

        def store(q, p):
            return pltpu.make_async_copy(
                bufs.at[pl.ds(q * 2 * GATHER_ROWS, 2 * GATHER_ROWS)],
                out_hbm.at[pl.ds(base + p * 2 * GATHER_ROWS,
                                 2 * GATHER_ROWS)],
                ssem[q])

        def fire_pair(q, p):
            gather(2 * q, 2 * p).start()
            gather(2 * q + 1, 2 * p + 1).start()

        def drain_pair(q, p):
            gather(2 * q, 2 * p).wait()
            gather(2 * q + 1, 2 * p + 1).wait()

        fire_pair(0, 0)

        def body(i, carry):
            for u in range(2):
                p = i * 2 + u
                q = u
                @pl.when(p + 1 < pairs)
                def _():
                    @pl.when(p >= 1)
                    def _():
                        store(1 - q, p - 1).wait()

                    fire_pair(1 - q, p + 1)

                drain_pair(q, p)
                store(q, p).start()
            return carry

        lax.fori_loop(0, pairs // 2, body, 0, unroll=False)

        store(0, pairs - 2).wait()
        store(1, pairs - 1).wait()

    return k


def kernel(sensor_ids, weight):
    b, t = sensor_ids.shape
    num_rows = b * t
    ids = sensor_ids.astype(jnp.int32).reshape(
        NUM_WORKERS, num_rows // (NUM_WORKERS * GATHER_ROWS), GATHER_ROWS)
    w_pad = jnp.pad(weight, ((0, TABLE_PAD - weight.shape[0]), (0, 0)))
    out = _make_sc_gather(num_rows)(ids, w_pad)
    return out.reshape(b, t, EMBED_D)


# speedup vs baseline: 1.0131x; 1.0047x over previous
import functools

import jax
import jax.numpy as jnp
from jax import lax
from jax.experimental import pallas as pl
from jax.experimental.pallas import tpu as pltpu
from jax.experimental.pallas import tpu_sc as plsc

EMBED_D = 128
NUM_WORKERS = 32
GATHER_ROWS = 128
TABLE_PAD = 1024
NBUF = 5
DEPTH = 3


def _make_sc_gather(num_rows: int):
    rows_per_w = num_rows // NUM_WORKERS
    chunks = rows_per_w // GATHER_ROWS
    assert chunks % NBUF == 0
    slab = TABLE_PAD // 16

    mesh = plsc.VectorSubcoreMesh(core_axis_name="c", subcore_axis_name="s")

    @functools.partial(
        pl.kernel,
        mesh=mesh,
        out_type=jax.ShapeDtypeStruct((num_rows, EMBED_D), jnp.float32),
        scratch_types=[
            pltpu.VMEM_SHARED((TABLE_PAD, EMBED_D), jnp.float32),
            pltpu.VMEM((chunks, GATHER_ROWS), jnp.int32),
        ]
        + [pltpu.VMEM((GATHER_ROWS, EMBED_D), jnp.float32)] * NBUF
        + [pltpu.SemaphoreType.DMA] * (2 * NBUF),
    )
    def k(ids_hbm, w_hbm, out_hbm, table_sh, idx_v, *rest):
        bufs = rest[:NBUF]
        gsem = rest[NBUF:2 * NBUF]
        ssem = rest[2 * NBUF:]
        cid = lax.axis_index("c")
        sid = lax.axis_index("s")
        wid = sid * 2 + cid
        base = wid * rows_per_w

        tcopy = pltpu.make_async_copy(w_hbm.at[pl.ds(sid * slab, slab)],
                                      table_sh.at[pl.ds(sid * slab, slab)],
                                      gsem[0])
        icopy = pltpu.make_async_copy(ids_hbm.at[wid], idx_v, gsem[1])
        tcopy.start()
        icopy.start()
        tcopy.wait()
        icopy.wait()
        plsc.subcore_barrier()

        def gather(b, j):
            return pltpu.make_async_copy(table_sh.at[idx_v.at[j]],
                                         bufs[b], gsem[b])

        def store(b, j):
            return pltpu.make_async_copy(
                bufs[b], out_hbm.at[pl.ds(base + j * GATHER_ROWS,
                                          GATHER_ROWS)], ssem[b])

        for p in range(DEPTH):
            gather(p, p).start()

        def body(i, carry):
            for u in range(NBUF):
                j = i * NBUF + u
                b = u
                gather(b, j).wait()
                store(b, j).start()
                b2 = (u + DEPTH) % NBUF

                @pl.when(j + DEPTH < chunks)
                def _():
                    @pl.when(j >= NBUF - DEPTH)
                    def _():
                        store(b2, j + DEPTH - NBUF).wait()

                    gather(b2, j + DEPTH).start()

            return carry

        lax.fori_loop(0, chunks // NBUF, body, 0, unroll=False)

        for u in range(NBUF):
            j = chunks - NBUF + u
            store(u, j).wait()

    return k


def kernel(sensor_ids, weight):
    b, t = sensor_ids.shape
    num_rows = b * t
    ids = sensor_ids.astype(jnp.int32).reshape(
        NUM_WORKERS, num_rows // (NUM_WORKERS * GATHER_ROWS), GATHER_ROWS)
    w_pad = jnp.pad(weight, ((0, TABLE_PAD - weight.shape[0]), (0, 0)))
    out = _make_sc_gather(num_rows)(ids, w_pad)
    return out.reshape(b, t, EMBED_D)
